# baseline (speedup 1.0000x reference)
import functools

import numpy as np

import jax
import jax.numpy as jnp
from jax import lax
from jax.experimental import pallas as pl
from jax.experimental.pallas import tpu as pltpu
from jax.experimental.pallas import tpu_sc as plsc

_LANES = 16


def _argmax_body(x_ref, i_ref, o_ref):
    x = x_ref[...]
    m = jnp.max(x, axis=2, keepdims=True)
    ii = i_ref[...]
    cand = jnp.where(x == m, ii, jnp.float32(x_ref.shape[2]))
    o_ref[...] = jnp.min(cand, axis=2)


def _make_hist(n, n_clus, n_clas):
    info = plsc.get_sparse_core_info()
    ns = info.num_subcores
    chunk = n // ns
    rows = n_clus // ns
    lpc = n_clas // _LANES
    unroll = 4
    inv_n = 1.0 / n
    assert chunk % (_LANES * unroll) == 0 and chunk % 8 == 0
    mesh = plsc.VectorSubcoreMesh(
        core_axis_name="c", subcore_axis_name="s", num_cores=1)

    @functools.partial(
        pl.kernel,
        out_type=jax.ShapeDtypeStruct((_LANES,), jnp.float32),
        mesh=mesh,
        scratch_types=[
            pltpu.VMEM((chunk,), jnp.float32),
            pltpu.VMEM((chunk,), jnp.int32),
            pltpu.VMEM((n_clus, n_clas), jnp.float32),
            pltpu.VMEM((rows, n_clas), jnp.float32),
            pltpu.VMEM((rows, n_clas), jnp.float32),
            pltpu.VMEM((_LANES, _LANES), jnp.float32),
            pltpu.VMEM((_LANES,), jnp.float32),
            pltpu.VMEM_SHARED((ns, n_clus, n_clas), jnp.float32),
            pltpu.VMEM_SHARED((_LANES, _LANES), jnp.float32),
        ],
        compiler_params=pltpu.CompilerParams(needs_layout_passes=False),
    )
    def hist(clus_hbm, tgt_hbm, out_hbm,
             clus_v, tgt_v, tab, acc, tmp, pvm, outv, shtab, shpart):
        sid = lax.axis_index("s")
        base = sid * chunk
        pltpu.sync_copy(clus_hbm.at[pl.ds(base, chunk)], clus_v)
        pltpu.sync_copy(tgt_hbm.at[pl.ds(base, chunk)], tgt_v)

        zeros = jnp.zeros((_LANES,), jnp.float32)

        def zero_step(i, carry):
            for j in range(lpc):
                tab[i, pl.ds(j * _LANES, _LANES)] = zeros
            return carry

        lax.fori_loop(0, n_clus, zero_step, 0)

        ones = jnp.ones((_LANES,), jnp.float32)

        def step(i, carry):
            for j in range(unroll):
                off = (i * unroll + j) * _LANES
                cv = clus_v[pl.ds(off, _LANES)].astype(jnp.int32)
                tv = tgt_v[pl.ds(off, _LANES)]
                plsc.addupdate_scatter(tab, [cv, tv], ones)
            return carry

        lax.fori_loop(0, chunk // (_LANES * unroll), step, 0)

        pltpu.sync_copy(tab, shtab.at[sid])
        plsc.subcore_barrier()

        r0 = sid * rows
        pltpu.sync_copy(shtab.at[0, pl.ds(r0, rows)], acc)
        for w in range(1, ns):
            pltpu.sync_copy(shtab.at[w, pl.ds(r0, rows)], tmp)
            for r in range(rows):
                for c in range(lpc):
                    s = pl.ds(c * _LANES, _LANES)
                    acc[r, s] = acc[r, s] + tmp[r, s]

        total = jnp.float32(0.0)
        for r in range(rows):
            m = acc[r, pl.ds(0, _LANES)]
            for c in range(1, lpc):
                m = jnp.maximum(m, acc[r, pl.ds(c * _LANES, _LANES)])
            total = total + jnp.max(m)

        outv[...] = jnp.full((_LANES,), total, jnp.float32)
        pltpu.sync_copy(outv, shpart.at[sid])
        plsc.subcore_barrier()

        @pl.when(sid == 0)
        def _():
            pltpu.sync_copy(shpart, pvm)
            p = pvm[0, pl.ds(0, _LANES)]
            for w in range(1, ns):
                p = p + pvm[w, pl.ds(0, _LANES)]
            outv[...] = p * inv_n
            pltpu.sync_copy(outv, out_hbm)

    return hist


def kernel(inputs, targets):
    n, n_clus = inputs.shape
    n_clas = 128
    row_blk = 64

    x3 = inputs.reshape(n // n_clus, n_clus, n_clus)
    lane_idx = jnp.asarray(
        np.arange(n_clus, dtype=np.float32).reshape(1, 1, n_clus))
    clus = pl.pallas_call(
        _argmax_body,
        grid=(x3.shape[0] // row_blk,),
        in_specs=[
            pl.BlockSpec((row_blk, n_clus, n_clus), lambda i: (i, 0, 0)),
            pl.BlockSpec((1, 1, n_clus), lambda i: (0, 0, 0)),
        ],
        out_specs=pl.BlockSpec((row_blk, n_clus), lambda i: (i, 0)),
        out_shape=jax.ShapeDtypeStruct((n // n_clus, n_clus), jnp.float32),
    )(x3, lane_idx)

    hist = _make_hist(n, n_clus, n_clas)
    out = hist(clus.reshape(n), targets)
    return out[0:1]


# speedup vs baseline: 1.0594x; 1.0594x over previous
import functools

import numpy as np

import jax
import jax.numpy as jnp
from jax import lax
from jax.experimental import pallas as pl
from jax.experimental.pallas import tpu as pltpu
from jax.experimental.pallas import tpu_sc as plsc

_LANES = 16


def _argmax_body(x_ref, i_ref, o_ref):
    x = x_ref[...]
    m = jnp.max(x, axis=2, keepdims=True)
    ii = i_ref[...]
    cand = jnp.where(x == m, ii, jnp.float32(x_ref.shape[2]))
    o_ref[...] = jnp.min(cand, axis=2)


def _finish_body(a_ref, b_ref, o_ref, inv_n):
    t = jnp.sum(a_ref[...], axis=0) + jnp.sum(b_ref[...], axis=0)
    m = jnp.max(t, axis=1, keepdims=True)
    o_ref[...] = jnp.sum(m, axis=0, keepdims=True) * inv_n


def _make_hist(nh, n_clus, n_clas, tgt_off):
    info = plsc.get_sparse_core_info()
    ns = info.num_subcores
    chunk = nh // ns
    unroll = 4
    assert chunk % (_LANES * unroll) == 0 and chunk % 8 == 0
    mesh = plsc.VectorSubcoreMesh(
        core_axis_name="c", subcore_axis_name="s", num_cores=1)

    @functools.partial(
        pl.kernel,
        out_type=jax.ShapeDtypeStruct((ns, n_clus, n_clas), jnp.float32),
        mesh=mesh,
        scratch_types=[
            pltpu.VMEM((chunk,), jnp.float32),
            pltpu.VMEM((chunk,), jnp.int32),
            pltpu.VMEM((n_clus, n_clas), jnp.float32),
        ],
        compiler_params=pltpu.CompilerParams(needs_layout_passes=False),
    )
    def hist(clus_hbm, tgt_hbm, out_hbm, clus_v, tgt_v, tab):
        wid = lax.axis_index("s")
        base = wid * chunk
        pltpu.sync_copy(clus_hbm.at[pl.ds(base, chunk)], clus_v)
        pltpu.sync_copy(tgt_hbm.at[pl.ds(tgt_off + base, chunk)], tgt_v)

        zeros = jnp.zeros((_LANES,), jnp.float32)

        def zero_step(i, carry):
            for j in range(n_clas // _LANES):
                tab[i, pl.ds(j * _LANES, _LANES)] = zeros
            return carry

        lax.fori_loop(0, n_clus, zero_step, 0)

        ones = jnp.ones((_LANES,), jnp.float32)

        def step(i, carry):
            for j in range(unroll):
                off = (i * unroll + j) * _LANES
                cv = clus_v[pl.ds(off, _LANES)].astype(jnp.int32)
                tv = tgt_v[pl.ds(off, _LANES)]
                plsc.addupdate_scatter(tab, [cv, tv], ones)
            return carry

        lax.fori_loop(0, chunk // (_LANES * unroll), step, 0)
        pltpu.sync_copy(tab, out_hbm.at[wid])

    return hist, ns


def kernel(inputs, targets):
    n, n_clus = inputs.shape
    n_clas = 128
    row_blk = 64
    halves = 2
    nh = n // halves

    x3 = inputs.reshape(n // n_clus, n_clus, n_clus)
    lane_idx = jnp.asarray(
        np.arange(n_clus, dtype=np.float32).reshape(1, 1, n_clus))
    steps = (nh // n_clus) // row_blk

    tabs = []
    for h in range(halves):
        clus_h = pl.pallas_call(
            _argmax_body,
            grid=(steps,),
            in_specs=[
                pl.BlockSpec((row_blk, n_clus, n_clus),
                             lambda i, h=h: (h * steps + i, 0, 0)),
                pl.BlockSpec((1, 1, n_clus), lambda i: (0, 0, 0)),
            ],
            out_specs=pl.BlockSpec((row_blk, n_clus), lambda i: (i, 0)),
            out_shape=jax.ShapeDtypeStruct((nh // n_clus, n_clus),
                                           jnp.float32),
        )(x3, lane_idx)
        hist, _ = _make_hist(nh, n_clus, n_clas, h * nh)
        tabs.append(hist(clus_h.reshape(nh), targets))

    out = pl.pallas_call(
        functools.partial(_finish_body, inv_n=1.0 / n),
        out_shape=jax.ShapeDtypeStruct((1, 1), jnp.float32),
    )(tabs[0], tabs[1])
    return out.reshape(1)
